# Initial kernel scaffold; baseline (speedup 1.0000x reference)
import jax
import jax.numpy as jnp
from jax.experimental import pallas as pl


def kernel(logits, mask):
    raise NotImplementedError("write your pallas kernel here")


# speedup vs baseline: 15.4769x; 15.4769x over previous
import functools

import jax
import jax.numpy as jnp
from jax import lax
from jax.experimental import pallas as pl
from jax.experimental.pallas import tpu as pltpu
from jax.experimental.pallas import tpu_sc as plsc

NCLS = 19
STRIDE = 32
CM_BINS = NCLS * STRIDE
TOT_BINS = 2 * CM_BINS
LANES = 16
NW = 32
N_PIX = 8 * 512 * 512
N_PER_W = N_PIX // NW
CHUNK = 32768
R_BLK = 128


def _bins_body(lref, mref, oref):
    x = lref[0]
    best = x[0]
    idx = jnp.zeros(best.shape, jnp.int32)
    for c in range(1, NCLS):
        v = x[c]
        upd = v > best
        best = jnp.where(upd, v, best)
        idx = jnp.where(upd, c, idx)
    oref[0] = mref[0] * STRIDE + idx


def _compute_bins(logits, mask):
    grid = (8, 512 // R_BLK)
    return pl.pallas_call(
        _bins_body,
        grid=grid,
        in_specs=[
            pl.BlockSpec((1, NCLS, R_BLK, 512), lambda b, r: (b, 0, r, 0)),
            pl.BlockSpec((1, R_BLK, 512), lambda b, r: (b, r, 0)),
        ],
        out_specs=pl.BlockSpec((1, R_BLK, 512), lambda b, r: (b, r, 0)),
        out_shape=jax.ShapeDtypeStruct((8, 512, 512), jnp.int32),
    )(logits, mask)


def _sc_hist_body(bins_hbm, out_hbm, buf, hist, outv):
    wid = lax.axis_index("s") * 2 + lax.axis_index("c")
    lane = lax.iota(jnp.int32, LANES)
    ones = jnp.ones((LANES,), jnp.float32)
    zeros = jnp.zeros((LANES,), jnp.float32)

    def zbody(i, _):
        hist[pl.ds(i * LANES, LANES)] = zeros
        return 0
    lax.fori_loop(0, TOT_BINS, zbody, 0)

    def chunk_body(ci, _):
        base = wid * N_PER_W + ci * CHUNK
        pltpu.sync_copy(bins_hbm.at[pl.ds(base, CHUNK)], buf)

        def inner(i, _):
            v = buf[pl.ds(i * LANES, LANES)]
            i1 = v * LANES + lane
            plsc.addupdate_scatter(hist, [i1], ones)
            p = lax.bitwise_and(v, STRIDE - 1)
            i2 = p * (33 * LANES) + (CM_BINS * LANES + lane)
            plsc.addupdate_scatter(hist, [i2], ones)
            return 0
        lax.fori_loop(0, CHUNK // LANES, inner, 0)
        return 0
    lax.fori_loop(0, N_PER_W // CHUNK, chunk_body, 0)

    def red_body(b0, _):
        base = b0 * (LANES * LANES)
        s = jnp.zeros((LANES,), jnp.float32)
        for l in range(LANES):
            s = s + plsc.load_gather(hist, [base + lane * LANES + l])
        outv[pl.ds(b0 * LANES, LANES)] = s
        return 0
    lax.fori_loop(0, TOT_BINS // LANES, red_body, 0)

    pltpu.sync_copy(outv, out_hbm.at[wid])


def _sc_hist(bins_flat):
    mesh = plsc.VectorSubcoreMesh(core_axis_name="c", subcore_axis_name="s")
    f = functools.partial(
        pl.kernel,
        mesh=mesh,
        out_type=jax.ShapeDtypeStruct((NW, TOT_BINS), jnp.float32),
        scratch_types=[
            pltpu.VMEM((CHUNK,), jnp.int32),
            pltpu.VMEM((TOT_BINS * LANES,), jnp.float32),
            pltpu.VMEM((TOT_BINS,), jnp.float32),
        ],
        compiler_params=pltpu.CompilerParams(needs_layout_passes=False),
    )(_sc_hist_body)
    return f(bins_flat)


def _stats_body(href, mou, fou):
    acc = href[0]
    for w in range(1, NW):
        acc = acc + href[w]
    cm = acc[0]
    pd = acc[1]
    rows = lax.broadcasted_iota(jnp.int32, (NCLS, STRIDE), 0)
    cols = lax.broadcasted_iota(jnp.int32, (NCLS, STRIDE), 1)
    eye = rows == cols
    diag = jnp.sum(jnp.where(eye, cm, 0.0), axis=1, keepdims=True)
    rowsum = jnp.sum(cm, axis=1, keepdims=True)
    colsum = jnp.sum(pd, axis=1, keepdims=True)
    total = jnp.sum(cm)
    denom = rowsum + colsum - diag
    dpos = denom > 0
    iu = jnp.where(dpos, diag / jnp.where(dpos, denom, 1.0), 0.0)
    miou = jnp.sum(iu) / NCLS
    freq = rowsum / jnp.where(total > 0, total, 1.0)
    fwiou = jnp.sum(jnp.where(freq > 0, freq * iu, 0.0))
    mou[...] = miou.reshape(1, 1)
    fou[...] = fwiou.reshape(1, 1)


def _compute_stats(hparts):
    return pl.pallas_call(
        _stats_body,
        out_shape=[
            jax.ShapeDtypeStruct((1, 1), jnp.float32),
            jax.ShapeDtypeStruct((1, 1), jnp.float32),
        ],
    )(hparts)


def kernel(logits, mask):
    bins = _compute_bins(logits, mask)
    hist = _sc_hist(bins.reshape(N_PIX))
    miou, fwiou = _compute_stats(hist.reshape(NW, 2, NCLS, STRIDE))
    return (miou.reshape(()), fwiou.reshape(()))


# speedup vs baseline: 17.4081x; 1.1248x over previous
import functools

import jax
import jax.numpy as jnp
from jax import lax
from jax.experimental import pallas as pl
from jax.experimental.pallas import tpu as pltpu
from jax.experimental.pallas import tpu_sc as plsc

NCLS = 19
STRIDE = 32
CM_BINS = NCLS * STRIDE
LANES = 16
NW = 32
N_PIX = 8 * 512 * 512
N_PER_W = N_PIX // NW
CHUNK = 32768
UNROLL = 8
R_BLK = 128


def _bins_body(lref, mref, oref):
    x = lref[0]
    best = x[0]
    idx = jnp.zeros(best.shape, jnp.int32)
    for c in range(1, NCLS):
        v = x[c]
        upd = v > best
        best = jnp.where(upd, v, best)
        idx = jnp.where(upd, c, idx)
    oref[0] = mref[0] * STRIDE + idx


def _compute_bins(logits, mask):
    grid = (8, 512 // R_BLK)
    return pl.pallas_call(
        _bins_body,
        grid=grid,
        in_specs=[
            pl.BlockSpec((1, NCLS, R_BLK, 512), lambda b, r: (b, 0, r, 0)),
            pl.BlockSpec((1, R_BLK, 512), lambda b, r: (b, r, 0)),
        ],
        out_specs=pl.BlockSpec((1, R_BLK, 512), lambda b, r: (b, r, 0)),
        out_shape=jax.ShapeDtypeStruct((8, 512, 512), jnp.int32),
    )(logits, mask)


def _sc_hist_body(bins_hbm, out_hbm, buf0, buf1, hist, outv, sem0, sem1):
    wid = lax.axis_index("s") * 2 + lax.axis_index("c")
    lane = lax.iota(jnp.int32, LANES)
    lane_base = lane * CM_BINS
    ones = jnp.ones((LANES,), jnp.float32)
    zeros = jnp.zeros((LANES,), jnp.float32)

    def zbody(i, _):
        b = i * (LANES * UNROLL)
        for u in range(UNROLL):
            hist[pl.ds(b + u * LANES, LANES)] = zeros
        return 0
    lax.fori_loop(0, (LANES * CM_BINS) // (LANES * UNROLL), zbody, 0)

    base = wid * N_PER_W
    cp0 = pltpu.async_copy(bins_hbm.at[pl.ds(base, CHUNK)], buf0, sem0)
    cp1 = pltpu.async_copy(bins_hbm.at[pl.ds(base + CHUNK, CHUNK)], buf1, sem1)

    def process(buf):
        def inner(i, _):
            b = i * (LANES * UNROLL)
            for u in range(UNROLL):
                v = buf[pl.ds(b + u * LANES, LANES)]
                plsc.addupdate_scatter(hist, [v + lane_base], ones)
            return 0
        lax.fori_loop(0, CHUNK // (LANES * UNROLL), inner, 0)

    cp0.wait()
    process(buf0)
    cp1.wait()
    process(buf1)

    def red_body(b0, _):
        s = jnp.zeros((LANES,), jnp.float32)
        for l in range(LANES):
            s = s + hist[pl.ds(l * CM_BINS + b0 * LANES, LANES)]
        outv[pl.ds(b0 * LANES, LANES)] = s
        return 0
    lax.fori_loop(0, CM_BINS // LANES, red_body, 0)

    pltpu.sync_copy(outv, out_hbm.at[wid])


def _sc_hist(bins_flat):
    mesh = plsc.VectorSubcoreMesh(core_axis_name="c", subcore_axis_name="s")
    f = functools.partial(
        pl.kernel,
        mesh=mesh,
        out_type=jax.ShapeDtypeStruct((NW, CM_BINS), jnp.float32),
        scratch_types=[
            pltpu.VMEM((CHUNK,), jnp.int32),
            pltpu.VMEM((CHUNK,), jnp.int32),
            pltpu.VMEM((LANES * CM_BINS,), jnp.float32),
            pltpu.VMEM((CM_BINS,), jnp.float32),
            pltpu.SemaphoreType.DMA,
            pltpu.SemaphoreType.DMA,
        ],
        compiler_params=pltpu.CompilerParams(needs_layout_passes=False),
    )(_sc_hist_body)
    return f(bins_flat)


def _stats_body(href, mou, fou):
    acc = href[0]
    for w in range(1, NW):
        acc = acc + href[w]
    cm = acc
    rows = lax.broadcasted_iota(jnp.int32, (NCLS, STRIDE), 0)
    cols = lax.broadcasted_iota(jnp.int32, (NCLS, STRIDE), 1)
    eye = rows == cols
    diag = jnp.sum(jnp.where(eye, cm, 0.0), axis=1, keepdims=True)
    rowsum = jnp.sum(cm, axis=1, keepdims=True)
    onescol = jnp.ones((NCLS, 1), jnp.float32)
    colsum32 = lax.dot_general(cm, onescol, (((0,), (0,)), ((), ())))
    colsum = colsum32[0:NCLS]
    total = jnp.sum(cm)
    denom = rowsum + colsum - diag
    dpos = denom > 0
    iu = jnp.where(dpos, diag / jnp.where(dpos, denom, 1.0), 0.0)
    miou = jnp.sum(iu) / NCLS
    freq = rowsum / jnp.where(total > 0, total, 1.0)
    fwiou = jnp.sum(jnp.where(freq > 0, freq * iu, 0.0))
    mou[...] = miou.reshape(1, 1)
    fou[...] = fwiou.reshape(1, 1)


def _compute_stats(hparts):
    return pl.pallas_call(
        _stats_body,
        out_shape=[
            jax.ShapeDtypeStruct((1, 1), jnp.float32),
            jax.ShapeDtypeStruct((1, 1), jnp.float32),
        ],
    )(hparts)


def kernel(logits, mask):
    bins = _compute_bins(logits, mask)
    hist = _sc_hist(bins.reshape(N_PIX))
    miou, fwiou = _compute_stats(hist.reshape(NW, NCLS, STRIDE))
    return (miou.reshape(()), fwiou.reshape(()))
